# baseline (speedup 1.0000x reference)
import functools

import jax
import jax.numpy as jnp
from jax import lax
from jax.experimental import pallas as pl
from jax.experimental.pallas import tpu as pltpu
from jax.experimental.pallas import tpu_sc as plsc

T = 2048
D = 1024
E = 8
I = 512
TB = 256
TB_SHIFT = 8
P_PAD = 2 * T + E * TB
NB = P_PAD // TB
SCALE = 2.5
NC, NS = 2, 16
NW = NC * NS
ROWS_W = P_PAD // NW
GCH = ROWS_W // 4
TOK_W = T // NW


def _shift_down(a, s):
    return jnp.concatenate([jnp.zeros((s, a.shape[1]), a.dtype), a[:-s]], 0)


def _shift_right(a, s):
    return jnp.concatenate([jnp.zeros((a.shape[0], s), a.dtype), a[:, :-s]], 1)


def _k1_body(x_ref, gw_ref, bias_ref,
             pos0_ref, pos1_ref, w0_ref, w1_ref, be_ref, bv_ref):
    x = x_ref[...]
    logits = lax.dot_general(x.astype(jnp.bfloat16),
                             gw_ref[...].astype(jnp.bfloat16),
                             (((1,), (1,)), ((), ())),
                             preferred_element_type=jnp.float32)
    scores = jax.nn.sigmoid(logits)
    s_choice = scores + bias_ref[...]
    col = lax.broadcasted_iota(jnp.int32, (T, E), 1)
    m1 = jnp.max(s_choice, 1, keepdims=True)
    i1 = jnp.min(jnp.where(s_choice >= m1, col, E), 1, keepdims=True)
    sel1 = col == i1
    s_excl = jnp.where(sel1, -1e30, s_choice)
    m2 = jnp.max(s_excl, 1, keepdims=True)
    i2 = jnp.min(jnp.where(s_excl >= m2, col, E), 1, keepdims=True)
    sel2 = col == i2
    wsum = jnp.sum(jnp.where(sel1 | sel2, scores, 0.0), 1, keepdims=True)
    w0 = jnp.sum(jnp.where(sel1, scores, 0.0), 1, keepdims=True) * (SCALE / wsum)
    w1 = jnp.sum(jnp.where(sel2, scores, 0.0), 1, keepdims=True) * (SCALE / wsum)
    w0_ref[...] = jnp.broadcast_to(w0, (T, 128))
    w1_ref[...] = jnp.broadcast_to(w1, (T, 128))

    f1 = sel1.astype(jnp.float32)
    f2 = sel2.astype(jnp.float32)
    c1, c2, s = f1, f2, 1
    while s < T:
        c1 = c1 + _shift_down(c1, s)
        c2 = c2 + _shift_down(c2, s)
        s *= 2
    tot0 = c1[T - 1:T, :]
    tot1 = c2[T - 1:T, :]
    C0 = c1 - f1
    C1 = c2 - f2
    counts = (tot0 + tot1).astype(jnp.int32)
    pc = ((counts + TB - 1) >> TB_SHIFT) << TB_SHIFT
    ic, s = pc, 1
    while s < E:
        ic = ic + _shift_right(ic, s)
        s *= 2
    po = ic - pc
    po_f = po.astype(jnp.float32)
    pos0 = jnp.sum(f1 * (po_f + C0), 1, keepdims=True)
    pos1 = jnp.sum(f2 * (po_f + tot0 + C1), 1, keepdims=True)
    pos0_ref[...] = pos0.astype(jnp.int32)
    pos1_ref[...] = pos1.astype(jnp.int32)

    brow = lax.broadcasted_iota(jnp.int32, (NB, E), 0) * TB
    colb = lax.broadcasted_iota(jnp.int32, (NB, E), 1)
    ind = (po <= brow) & (brow < po + pc)
    bv = jnp.sum(ind.astype(jnp.int32), 1, keepdims=True)
    be_ref[...] = jnp.sum(jnp.where(ind, colb, 0), 1, keepdims=True) + 7 * (1 - bv)
    bv_ref[...] = bv


def _router_dispatch(x, gate_w, bias):
    return pl.pallas_call(
        _k1_body,
        in_specs=[pl.BlockSpec((T, D), lambda: (0, 0)),
                  pl.BlockSpec((E, D), lambda: (0, 0)),
                  pl.BlockSpec((1, E), lambda: (0, 0))],
        out_specs=[pl.BlockSpec((T, 1), lambda: (0, 0)),
                   pl.BlockSpec((T, 1), lambda: (0, 0)),
                   pl.BlockSpec((T, 128), lambda: (0, 0)),
                   pl.BlockSpec((T, 128), lambda: (0, 0)),
                   pl.BlockSpec((NB, 1), lambda: (0, 0)),
                   pl.BlockSpec((NB, 1), lambda: (0, 0))],
        out_shape=[jax.ShapeDtypeStruct((T, 1), jnp.int32),
                   jax.ShapeDtypeStruct((T, 1), jnp.int32),
                   jax.ShapeDtypeStruct((T, 128), jnp.float32),
                   jax.ShapeDtypeStruct((T, 128), jnp.float32),
                   jax.ShapeDtypeStruct((NB, 1), jnp.int32),
                   jax.ShapeDtypeStruct((NB, 1), jnp.int32)],
    )(x, gate_w, bias)


@functools.cache
def _sc_mesh():
    return plsc.VectorSubcoreMesh(core_axis_name="c", subcore_axis_name="s",
                                  num_cores=NC, num_subcores=NS)


@functools.cache
def _k3_kernel():
    return pl.kernel(
        _k3_body,
        out_type=[jax.ShapeDtypeStruct((P_PAD, D), jnp.float32),
                  jax.ShapeDtypeStruct((P_PAD, 128), jnp.float32)],
        mesh=_sc_mesh(),
        scratch_types=[
            pltpu.VMEM((TOK_W,), jnp.int32),
            pltpu.VMEM((TOK_W,), jnp.int32),
            pltpu.VMEM((TOK_W, D), jnp.float32),
            pltpu.VMEM((TOK_W, 128), jnp.float32),
            pltpu.VMEM((TOK_W, 128), jnp.float32),
            pltpu.SemaphoreType.DMA,
            pltpu.SemaphoreType.DMA,
            pltpu.SemaphoreType.DMA,
            pltpu.SemaphoreType.DMA,
        ],
        compiler_params=pltpu.CompilerParams(needs_layout_passes=False),
    )


def _k3_body(pos0_hbm, pos1_hbm, w0_hbm, w1_hbm, x_hbm,
             xs_hbm, ws_hbm,
             idx0_v, idx1_v, xrows_v, wr0_v, wr1_v,
             sem0, sem1, sem2, sem3):
    wid = lax.axis_index("c") * NS + lax.axis_index("s")
    tbase = wid * TOK_W
    pltpu.sync_copy(pos0_hbm.at[pl.ds(tbase, TOK_W)], idx0_v)
    pltpu.sync_copy(pos1_hbm.at[pl.ds(tbase, TOK_W)], idx1_v)
    pltpu.sync_copy(x_hbm.at[pl.ds(tbase, TOK_W)], xrows_v)
    pltpu.sync_copy(w0_hbm.at[pl.ds(tbase, TOK_W)], wr0_v)
    pltpu.sync_copy(w1_hbm.at[pl.ds(tbase, TOK_W)], wr1_v)
    d0 = pltpu.async_copy(xrows_v, xs_hbm.at[idx0_v], sem0)
    d1 = pltpu.async_copy(xrows_v, xs_hbm.at[idx1_v], sem1)
    d2 = pltpu.async_copy(wr0_v, ws_hbm.at[idx0_v], sem2)
    d3 = pltpu.async_copy(wr1_v, ws_hbm.at[idx1_v], sem3)
    d0.wait()
    d1.wait()
    d2.wait()
    d3.wait()


@functools.cache
def _k5_kernel():
    return pl.kernel(
        _k5_body,
        out_type=[jax.ShapeDtypeStruct((T, D), jnp.float32),
                  jax.ShapeDtypeStruct((T, D), jnp.float32)],
        mesh=_sc_mesh(),
        scratch_types=[
            pltpu.VMEM((TOK_W,), jnp.int32),
            pltpu.VMEM((TOK_W,), jnp.int32),
            pltpu.VMEM((TOK_W // 2, D), jnp.float32),
            pltpu.VMEM((TOK_W // 2, D), jnp.float32),
            pltpu.SemaphoreType.DMA,
            pltpu.SemaphoreType.DMA,
        ],
        compiler_params=pltpu.CompilerParams(needs_layout_passes=False),
    )


def _k5_body(pos0_hbm, pos1_hbm, ys_hbm, y0_hbm, y1_hbm,
             idx0_v, idx1_v, rows0, rows1, sem0, sem1):
    wid = lax.axis_index("c") * NS + lax.axis_index("s")
    base = wid * TOK_W
    half = TOK_W // 2
    pltpu.sync_copy(pos0_hbm.at[pl.ds(base, TOK_W)], idx0_v)
    pltpu.sync_copy(pos1_hbm.at[pl.ds(base, TOK_W)], idx1_v)
    plan = ((idx0_v, 0, y0_hbm), (idx0_v, half, y0_hbm),
            (idx1_v, 0, y1_hbm), (idx1_v, half, y1_hbm))
    bufs = (rows0, rows1)
    sems = (sem0, sem1)
    descs = [None, None]
    outs = [None, None]
    for i, (idx, off, out_hbm) in enumerate(plan):
        b = i % 2
        if descs[b] is not None:
            descs[b].wait()
            pltpu.sync_copy(bufs[b], outs[b])
        descs[b] = pltpu.async_copy(
            ys_hbm.at[idx.at[pl.ds(off, half)]], bufs[b], sems[b])
        outs[b] = out_hbm.at[pl.ds(base + off, half)]
    for b in (0, 1):
        descs[b].wait()
        pltpu.sync_copy(bufs[b], outs[b])


def _bf16_mlp(x_bf, wg, wu, wd):
    g = lax.dot_general(x_bf, wg.astype(jnp.bfloat16),
                        (((1,), (1,)), ((), ())),
                        preferred_element_type=jnp.float32)
    u = lax.dot_general(x_bf, wu.astype(jnp.bfloat16),
                        (((1,), (1,)), ((), ())),
                        preferred_element_type=jnp.float32)
    h = (g * jax.nn.sigmoid(g) * u).astype(jnp.bfloat16)
    return lax.dot_general(h, wd.astype(jnp.bfloat16),
                           (((1,), (1,)), ((), ())),
                           preferred_element_type=jnp.float32)


NBX = NB + T // TB


def _k4_body(be_ref, bv_ref, x_ref, wg_ref, wu_ref, wd_ref, w_ref,
             xh_ref, sg_ref, su_ref, sd_ref, y_ref):
    b = pl.program_id(0)

    @pl.when((b < NB) & (bv_ref[jnp.minimum(b, NB - 1)] > 0))
    def _routed():
        y = _bf16_mlp(x_ref[...].astype(jnp.bfloat16),
                      wg_ref[0], wu_ref[0], wd_ref[0])
        y_ref[...] = y * w_ref[0][:, 0:1]

    @pl.when(b >= NB)
    def _shared():
        y_ref[...] = _bf16_mlp(xh_ref[...].astype(jnp.bfloat16),
                               sg_ref[...], su_ref[...], sd_ref[...])


def _expert_blocks(x_sorted, w_gate, w_up, w_down, w_sorted3d,
                   block_expert, block_valid, hidden, sg, su, sd):
    rb = lambda b: jnp.minimum(b, NB - 1)
    grid_spec = pltpu.PrefetchScalarGridSpec(
        num_scalar_prefetch=2,
        grid=(NBX,),
        in_specs=[
            pl.BlockSpec((TB, D), lambda b, be, bv: (rb(b), 0)),
            pl.BlockSpec((1, I, D), lambda b, be, bv: (be[rb(b)], 0, 0)),
            pl.BlockSpec((1, I, D), lambda b, be, bv: (be[rb(b)], 0, 0)),
            pl.BlockSpec((1, D, I), lambda b, be, bv: (be[rb(b)], 0, 0)),
            pl.BlockSpec((1, TB, 128), lambda b, be, bv: (rb(b), 0, 0)),
            pl.BlockSpec((TB, D),
                         lambda b, be, bv: (jnp.maximum(b - NB, 0), 0)),
            pl.BlockSpec((I, D), lambda b, be, bv: (0, 0)),
            pl.BlockSpec((I, D), lambda b, be, bv: (0, 0)),
            pl.BlockSpec((D, I), lambda b, be, bv: (0, 0)),
        ],
        out_specs=pl.BlockSpec((TB, D), lambda b, be, bv: (b, 0)),
    )
    return pl.pallas_call(
        _k4_body,
        grid_spec=grid_spec,
        out_shape=jax.ShapeDtypeStruct((NBX * TB, D), jnp.float32),
    )(block_expert, block_valid, x_sorted, w_gate, w_up, w_down, w_sorted3d,
      hidden, sg, su, sd)


TT6 = 512


def _k6a_body(x_ref, sg_ref, su_ref, sd_ref, out_ref):
    out_ref[...] = _bf16_mlp(x_ref[...].astype(jnp.bfloat16),
                             sg_ref[...], su_ref[...], sd_ref[...])


def _shared_mlp(x, sg, su, sd):
    return pl.pallas_call(
        _k6a_body,
        grid=(T // TT6,),
        in_specs=[pl.BlockSpec((TT6, D), lambda t: (t, 0)),
                  pl.BlockSpec((I, D), lambda t: (0, 0)),
                  pl.BlockSpec((I, D), lambda t: (0, 0)),
                  pl.BlockSpec((D, I), lambda t: (0, 0))],
        out_specs=pl.BlockSpec((TT6, D), lambda t: (t, 0)),
        out_shape=jax.ShapeDtypeStruct((T, D), jnp.float32),
    )(x, sg, su, sd)


def _k7_body(s_ref, y0_ref, y1_ref, out_ref):
    out_ref[...] = s_ref[...] + y0_ref[...] + y1_ref[...]


def _final_add(y_all, y0, y1):
    return pl.pallas_call(
        _k7_body,
        grid=(T // TT6,),
        in_specs=[pl.BlockSpec((TT6, D), lambda t: (P_PAD // TT6 + t, 0)),
                  pl.BlockSpec((TT6, D), lambda t: (t, 0)),
                  pl.BlockSpec((TT6, D), lambda t: (t, 0))],
        out_specs=pl.BlockSpec((TT6, D), lambda t: (t, 0)),
        out_shape=jax.ShapeDtypeStruct((T, D), jnp.float32),
    )(y_all, y0, y1)


@jax.jit
def kernel(hidden_states, gate_w, w_gate, w_up, w_down,
           shared_gate_w, shared_up_w, shared_down_w, correction_bias):
    bias = correction_bias.reshape(1, E).astype(jnp.float32)
    pos0, pos1, w0, w1, be, bv = _router_dispatch(hidden_states, gate_w, bias)
    pos0f = pos0.reshape(T)
    pos1f = pos1.reshape(T)
    x_sorted, w_sorted = _k3_kernel()(
        pos0f, pos1f, w0, w1, hidden_states)
    y_all = _expert_blocks(x_sorted, w_gate, w_up, w_down,
                           w_sorted.reshape(NB, TB, 128),
                           be.reshape(NB), bv.reshape(NB),
                           hidden_states, shared_gate_w, shared_up_w,
                           shared_down_w)
    y0, y1 = _k5_kernel()(pos0f, pos1f, y_all)
    return _final_add(y_all, y0, y1)


# speedup vs baseline: 1.0801x; 1.0801x over previous
import functools

import jax
import jax.numpy as jnp
from jax import lax
from jax.experimental import pallas as pl
from jax.experimental.pallas import tpu as pltpu
from jax.experimental.pallas import tpu_sc as plsc

T = 2048
D = 1024
E = 8
I = 512
TB = 256
TB_SHIFT = 8
P_PAD = 2 * T + E * TB
NB = P_PAD // TB
SCALE = 2.5
NC, NS = 2, 16
NW = NC * NS
ROWS_W = P_PAD // NW
GCH = ROWS_W // 4
TOK_W = T // NW


def _shift_down(a, s):
    return jnp.concatenate([jnp.zeros((s, a.shape[1]), a.dtype), a[:-s]], 0)


def _shift_right(a, s):
    return jnp.concatenate([jnp.zeros((a.shape[0], s), a.dtype), a[:, :-s]], 1)


def _k1_body(x_ref, gw_ref, bias_ref,
             pos0_ref, pos1_ref, w0_ref, w1_ref, be_ref, bv_ref):
    x = x_ref[...]
    logits = lax.dot_general(x.astype(jnp.bfloat16),
                             gw_ref[...].astype(jnp.bfloat16),
                             (((1,), (1,)), ((), ())),
                             preferred_element_type=jnp.float32)
    scores = jax.nn.sigmoid(logits)
    s_choice = scores + bias_ref[...]
    col = lax.broadcasted_iota(jnp.int32, (T, E), 1)
    m1 = jnp.max(s_choice, 1, keepdims=True)
    i1 = jnp.min(jnp.where(s_choice >= m1, col, E), 1, keepdims=True)
    sel1 = col == i1
    s_excl = jnp.where(sel1, -1e30, s_choice)
    m2 = jnp.max(s_excl, 1, keepdims=True)
    i2 = jnp.min(jnp.where(s_excl >= m2, col, E), 1, keepdims=True)
    sel2 = col == i2
    wsum = jnp.sum(jnp.where(sel1 | sel2, scores, 0.0), 1, keepdims=True)
    w0 = jnp.sum(jnp.where(sel1, scores, 0.0), 1, keepdims=True) * (SCALE / wsum)
    w1 = jnp.sum(jnp.where(sel2, scores, 0.0), 1, keepdims=True) * (SCALE / wsum)
    w0_ref[...] = jnp.broadcast_to(w0, (T, 128))
    w1_ref[...] = jnp.broadcast_to(w1, (T, 128))

    f1 = sel1.astype(jnp.float32)
    f2 = sel2.astype(jnp.float32)
    c1, c2, s = f1, f2, 1
    while s < T:
        c1 = c1 + _shift_down(c1, s)
        c2 = c2 + _shift_down(c2, s)
        s *= 2
    tot0 = c1[T - 1:T, :]
    tot1 = c2[T - 1:T, :]
    C0 = c1 - f1
    C1 = c2 - f2
    counts = (tot0 + tot1).astype(jnp.int32)
    pc = ((counts + TB - 1) >> TB_SHIFT) << TB_SHIFT
    ic, s = pc, 1
    while s < E:
        ic = ic + _shift_right(ic, s)
        s *= 2
    po = ic - pc
    po_f = po.astype(jnp.float32)
    pos0 = jnp.sum(f1 * (po_f + C0), 1, keepdims=True)
    pos1 = jnp.sum(f2 * (po_f + tot0 + C1), 1, keepdims=True)
    pos0_ref[...] = pos0.astype(jnp.int32)
    pos1_ref[...] = pos1.astype(jnp.int32)

    brow = lax.broadcasted_iota(jnp.int32, (NB, E), 0) * TB
    colb = lax.broadcasted_iota(jnp.int32, (NB, E), 1)
    ind = (po <= brow) & (brow < po + pc)
    bv = jnp.sum(ind.astype(jnp.int32), 1, keepdims=True)
    be_ref[...] = jnp.sum(jnp.where(ind, colb, 0), 1, keepdims=True) + 7 * (1 - bv)
    bv_ref[...] = bv


def _router_dispatch(x, gate_w, bias):
    return pl.pallas_call(
        _k1_body,
        in_specs=[pl.BlockSpec((T, D), lambda: (0, 0)),
                  pl.BlockSpec((E, D), lambda: (0, 0)),
                  pl.BlockSpec((1, E), lambda: (0, 0))],
        out_specs=[pl.BlockSpec((T, 1), lambda: (0, 0)),
                   pl.BlockSpec((T, 1), lambda: (0, 0)),
                   pl.BlockSpec((T, 128), lambda: (0, 0)),
                   pl.BlockSpec((T, 128), lambda: (0, 0)),
                   pl.BlockSpec((NB, 1), lambda: (0, 0)),
                   pl.BlockSpec((NB, 1), lambda: (0, 0))],
        out_shape=[jax.ShapeDtypeStruct((T, 1), jnp.int32),
                   jax.ShapeDtypeStruct((T, 1), jnp.int32),
                   jax.ShapeDtypeStruct((T, 128), jnp.float32),
                   jax.ShapeDtypeStruct((T, 128), jnp.float32),
                   jax.ShapeDtypeStruct((NB, 1), jnp.int32),
                   jax.ShapeDtypeStruct((NB, 1), jnp.int32)],
    )(x, gate_w, bias)


@functools.cache
def _sc_mesh():
    return plsc.VectorSubcoreMesh(core_axis_name="c", subcore_axis_name="s",
                                  num_cores=NC, num_subcores=NS)


@functools.cache
def _k3_kernel():
    return pl.kernel(
        _k3_body,
        out_type=[jax.ShapeDtypeStruct((P_PAD, D), jnp.float32),
                  jax.ShapeDtypeStruct((P_PAD, 128), jnp.float32)],
        mesh=_sc_mesh(),
        scratch_types=[
            pltpu.VMEM((TOK_W,), jnp.int32),
            pltpu.VMEM((TOK_W,), jnp.int32),
            pltpu.VMEM((TOK_W, D), jnp.float32),
            pltpu.VMEM((TOK_W, 128), jnp.float32),
            pltpu.VMEM((TOK_W, 128), jnp.float32),
            pltpu.SemaphoreType.DMA,
            pltpu.SemaphoreType.DMA,
            pltpu.SemaphoreType.DMA,
            pltpu.SemaphoreType.DMA,
        ],
        compiler_params=pltpu.CompilerParams(needs_layout_passes=False),
    )


def _k3_body(pos0_hbm, pos1_hbm, w0_hbm, w1_hbm, x_hbm,
             xs_hbm, ws_hbm,
             idx0_v, idx1_v, xrows_v, wr0_v, wr1_v,
             sem0, sem1, sem2, sem3):
    wid = lax.axis_index("c") * NS + lax.axis_index("s")
    tbase = wid * TOK_W
    pltpu.sync_copy(pos0_hbm.at[pl.ds(tbase, TOK_W)], idx0_v)
    pltpu.sync_copy(pos1_hbm.at[pl.ds(tbase, TOK_W)], idx1_v)
    pltpu.sync_copy(x_hbm.at[pl.ds(tbase, TOK_W)], xrows_v)
    pltpu.sync_copy(w0_hbm.at[pl.ds(tbase, TOK_W)], wr0_v)
    pltpu.sync_copy(w1_hbm.at[pl.ds(tbase, TOK_W)], wr1_v)
    d0 = pltpu.async_copy(xrows_v, xs_hbm.at[idx0_v], sem0)
    d1 = pltpu.async_copy(xrows_v, xs_hbm.at[idx1_v], sem1)
    d2 = pltpu.async_copy(wr0_v, ws_hbm.at[idx0_v], sem2)
    d3 = pltpu.async_copy(wr1_v, ws_hbm.at[idx1_v], sem3)
    d0.wait()
    d1.wait()
    d2.wait()
    d3.wait()


@functools.cache
def _k5_kernel():
    return pl.kernel(
        _k5_body,
        out_type=[jax.ShapeDtypeStruct((T, D), jnp.float32),
                  jax.ShapeDtypeStruct((T, D), jnp.float32)],
        mesh=_sc_mesh(),
        scratch_types=[
            pltpu.VMEM((TOK_W,), jnp.int32),
            pltpu.VMEM((TOK_W,), jnp.int32),
            pltpu.VMEM((TOK_W // 2, D), jnp.float32),
            pltpu.VMEM((TOK_W // 2, D), jnp.float32),
            pltpu.SemaphoreType.DMA,
            pltpu.SemaphoreType.DMA,
        ],
        compiler_params=pltpu.CompilerParams(needs_layout_passes=False),
    )


def _k5_body(pos0_hbm, pos1_hbm, ys_hbm, y0_hbm, y1_hbm,
             idx0_v, idx1_v, rows0, rows1, sem0, sem1):
    wid = lax.axis_index("c") * NS + lax.axis_index("s")
    base = wid * TOK_W
    half = TOK_W // 2
    pltpu.sync_copy(pos0_hbm.at[pl.ds(base, TOK_W)], idx0_v)
    pltpu.sync_copy(pos1_hbm.at[pl.ds(base, TOK_W)], idx1_v)
    plan = ((idx0_v, 0, y0_hbm), (idx0_v, half, y0_hbm),
            (idx1_v, 0, y1_hbm), (idx1_v, half, y1_hbm))
    bufs = (rows0, rows1)
    sems = (sem0, sem1)
    descs = [None, None]
    outs = [None, None]
    for i, (idx, off, out_hbm) in enumerate(plan):
        b = i % 2
        if descs[b] is not None:
            descs[b].wait()
            pltpu.sync_copy(bufs[b], outs[b])
        descs[b] = pltpu.async_copy(
            ys_hbm.at[idx.at[pl.ds(off, half)]], bufs[b], sems[b])
        outs[b] = out_hbm.at[pl.ds(base + off, half)]
    for b in (0, 1):
        descs[b].wait()
        pltpu.sync_copy(bufs[b], outs[b])


def _bf16_mlp(x_bf, wg, wu, wd):
    g = lax.dot_general(x_bf, wg.astype(jnp.bfloat16),
                        (((1,), (1,)), ((), ())),
                        preferred_element_type=jnp.float32)
    u = lax.dot_general(x_bf, wu.astype(jnp.bfloat16),
                        (((1,), (1,)), ((), ())),
                        preferred_element_type=jnp.float32)
    h = (g * jax.nn.sigmoid(g) * u).astype(jnp.bfloat16)
    return lax.dot_general(h, wd.astype(jnp.bfloat16),
                           (((1,), (1,)), ((), ())),
                           preferred_element_type=jnp.float32)


def _k4_body(be_ref, bv_ref, x_ref, wg_ref, wu_ref, wd_ref, w_ref, y_ref):
    @pl.when(bv_ref[pl.program_id(0)] > 0)
    def _():
        y = _bf16_mlp(x_ref[...].astype(jnp.bfloat16),
                      wg_ref[0], wu_ref[0], wd_ref[0])
        y_ref[...] = y * w_ref[0][:, 0:1]


def _expert_blocks(x_sorted, w_gate, w_up, w_down, w_sorted3d,
                   block_expert, block_valid):
    grid_spec = pltpu.PrefetchScalarGridSpec(
        num_scalar_prefetch=2,
        grid=(NB,),
        in_specs=[
            pl.BlockSpec((TB, D), lambda b, be, bv: (b, 0)),
            pl.BlockSpec((1, I, D), lambda b, be, bv: (be[b], 0, 0)),
            pl.BlockSpec((1, I, D), lambda b, be, bv: (be[b], 0, 0)),
            pl.BlockSpec((1, D, I), lambda b, be, bv: (be[b], 0, 0)),
            pl.BlockSpec((1, TB, 128), lambda b, be, bv: (b, 0, 0)),
        ],
        out_specs=pl.BlockSpec((TB, D), lambda b, be, bv: (b, 0)),
    )
    return pl.pallas_call(
        _k4_body,
        grid_spec=grid_spec,
        out_shape=jax.ShapeDtypeStruct((P_PAD, D), jnp.float32),
    )(block_expert, block_valid, x_sorted, w_gate, w_up, w_down, w_sorted3d)


TT6 = 512


def _k6a_body(x_ref, sg_ref, su_ref, sd_ref, out_ref):
    out_ref[...] = _bf16_mlp(x_ref[...].astype(jnp.bfloat16),
                             sg_ref[...], su_ref[...], sd_ref[...])


def _shared_mlp(x, sg, su, sd):
    return pl.pallas_call(
        _k6a_body,
        grid=(T // TT6,),
        in_specs=[pl.BlockSpec((TT6, D), lambda t: (t, 0)),
                  pl.BlockSpec((I, D), lambda t: (0, 0)),
                  pl.BlockSpec((I, D), lambda t: (0, 0)),
                  pl.BlockSpec((D, I), lambda t: (0, 0))],
        out_specs=pl.BlockSpec((TT6, D), lambda t: (t, 0)),
        out_shape=jax.ShapeDtypeStruct((T, D), jnp.float32),
    )(x, sg, su, sd)


def _k7_body(s_ref, y0_ref, y1_ref, out_ref):
    out_ref[...] = s_ref[...] + y0_ref[...] + y1_ref[...]


def _final_add(shared_out, y0, y1):
    return pl.pallas_call(
        _k7_body,
        grid=(T // TT6,),
        in_specs=[pl.BlockSpec((TT6, D), lambda t: (t, 0)),
                  pl.BlockSpec((TT6, D), lambda t: (t, 0)),
                  pl.BlockSpec((TT6, D), lambda t: (t, 0))],
        out_specs=pl.BlockSpec((TT6, D), lambda t: (t, 0)),
        out_shape=jax.ShapeDtypeStruct((T, D), jnp.float32),
    )(shared_out, y0, y1)


@jax.jit
def kernel(hidden_states, gate_w, w_gate, w_up, w_down,
           shared_gate_w, shared_up_w, shared_down_w, correction_bias):
    bias = correction_bias.reshape(1, E).astype(jnp.float32)
    pos0, pos1, w0, w1, be, bv = _router_dispatch(hidden_states, gate_w, bias)
    pos0f = pos0.reshape(T)
    pos1f = pos1.reshape(T)
    x_sorted, w_sorted = _k3_kernel()(
        pos0f, pos1f, w0, w1, hidden_states)
    y_sorted = _expert_blocks(x_sorted, w_gate, w_up, w_down,
                              w_sorted.reshape(NB, TB, 128),
                              be.reshape(NB), bv.reshape(NB))
    shared_out = _shared_mlp(hidden_states, shared_gate_w, shared_up_w,
                             shared_down_w)
    y0, y1 = _k5_kernel()(pos0f, pos1f, y_sorted)
    return _final_add(shared_out, y0, y1)
